# baseline (speedup 1.0000x reference)
import functools

import jax
import jax.numpy as jnp
from jax import lax
from jax.experimental import pallas as pl
from jax.experimental.pallas import tpu as pltpu
from jax.experimental.pallas import tpu_sc as plsc

N = 10000
NP = 10240
D = 128
H = 64
K = 32
E = 320000
NTILES = 32
CHK = 1024
NCH = 10
EPT = CHK * NCH
EP = EPT * NTILES
RPT = NP // 16

_f32 = jnp.float32


def _matmul_call(x, w):
    def body(x_ref, w_ref, o_ref):
        o_ref[...] = jnp.dot(x_ref[...], w_ref[...],
                             preferred_element_type=_f32)

    return pl.pallas_call(
        body,
        out_shape=jax.ShapeDtypeStruct((x.shape[0], w.shape[1]), _f32),
    )(x, w)


def _segsum_sc(g, src3, dst3, zblk):
    mesh = plsc.VectorSubcoreMesh(core_axis_name="c", subcore_axis_name="s")

    @functools.partial(
        pl.kernel,
        out_type=jax.ShapeDtypeStruct((2, NP, H), _f32),
        mesh=mesh,
        compiler_params=pltpu.CompilerParams(use_tc_tiling_on_sc=False),
        scratch_types=[
            pltpu.VMEM((CHK,), jnp.int32),
            pltpu.VMEM((CHK,), jnp.int32),
            pltpu.VMEM((CHK, H), _f32),
            pltpu.VMEM_SHARED((NP, H), _f32),
        ],
    )
    def ksc(g_hbm, src_hbm, dst_hbm, z_hbm, out_hbm,
            srcv, dstv, rows, acc):
        c = lax.axis_index("c")
        s = lax.axis_index("s")
        wid = s * 2 + c

        pltpu.sync_copy(z_hbm, rows.at[pl.ds(0, RPT)])
        pltpu.sync_copy(rows.at[pl.ds(0, RPT)], acc.at[pl.ds(s * RPT, RPT)])
        plsc.subcore_barrier()

        @pl.loop(0, NCH)
        def _(j):
            pltpu.sync_copy(src_hbm.at[wid, j], srcv)
            pltpu.sync_copy(dst_hbm.at[wid, j], dstv)
            pltpu.sync_copy(g_hbm.at[srcv], rows)
            pltpu.sync_copy(rows, acc.at[dstv], add=True)

        plsc.subcore_barrier()
        pltpu.sync_copy(acc.at[pl.ds(s * RPT, RPT)], rows.at[pl.ds(0, RPT)])
        pltpu.sync_copy(rows.at[pl.ds(0, RPT)], out_hbm.at[c, pl.ds(s * RPT, RPT)])

    return ksc(g, src3, dst3, zblk)


def _topk_pool(s80, hs_ref, r_ref):
    ia = lax.broadcasted_iota(jnp.int32, (80, 128), 0)
    ib = lax.broadcasted_iota(jnp.int32, (80, 128), 1)
    fiota = ia * 128 + ib
    s = jnp.where(fiota < N, s80, _f32(-3e38))

    def step(k, sv):
        m = jnp.max(sv)
        fi = jnp.min(jnp.where(sv == m, fiota, jnp.int32(1 << 30)))
        row = hs_ref[pl.ds(fi, 1), :]
        r_ref[pl.ds(k, 1), :] = row * m
        return jnp.where(fiota == fi, _f32(-3e38), sv)

    lax.fori_loop(0, K, step, s)


def _combine_call(g, p, scal, b, aw, wn):
    def body(scal_ref, g_ref, p_ref, b_ref, aw_ref, wn_ref,
             gn_ref, r_ref, hs_ref):
        one_eps = scal_ref[0]
        ab = scal_ref[1]
        h = jnp.maximum(
            one_eps * g_ref[...] + p_ref[0] + p_ref[1] + b_ref[...], 0.0)
        hs_ref[...] = h
        gn_ref[...] = jnp.dot(h, wn_ref[...], preferred_element_type=_f32)
        z = jnp.dot(h, aw_ref[...], preferred_element_type=_f32)
        s80 = jnp.tanh(z + ab).reshape(80, 128)
        _topk_pool(s80, hs_ref, r_ref)

    return pl.pallas_call(
        body,
        in_specs=[
            pl.BlockSpec(memory_space=pltpu.SMEM),
            pl.BlockSpec(memory_space=pltpu.VMEM),
            pl.BlockSpec(memory_space=pltpu.VMEM),
            pl.BlockSpec(memory_space=pltpu.VMEM),
            pl.BlockSpec(memory_space=pltpu.VMEM),
            pl.BlockSpec(memory_space=pltpu.VMEM),
        ],
        out_shape=[
            jax.ShapeDtypeStruct((NP, H), _f32),
            jax.ShapeDtypeStruct((K, H), _f32),
        ],
        scratch_shapes=[pltpu.VMEM((NP, H), _f32)],
    )(scal, g, p, b, aw, wn)


def _combine_last_call(g, p, scal, b, aw):
    def body(scal_ref, g_ref, p_ref, b_ref, aw_ref, r_ref, hs_ref):
        one_eps = scal_ref[0]
        ab = scal_ref[1]
        h = jnp.maximum(
            one_eps * g_ref[...] + p_ref[0] + p_ref[1] + b_ref[...], 0.0)
        hs_ref[...] = h
        z = jnp.dot(h, aw_ref[...], preferred_element_type=_f32)
        s80 = jnp.tanh(z + ab).reshape(80, 128)
        _topk_pool(s80, hs_ref, r_ref)

    return pl.pallas_call(
        body,
        in_specs=[
            pl.BlockSpec(memory_space=pltpu.SMEM),
            pl.BlockSpec(memory_space=pltpu.VMEM),
            pl.BlockSpec(memory_space=pltpu.VMEM),
            pl.BlockSpec(memory_space=pltpu.VMEM),
            pl.BlockSpec(memory_space=pltpu.VMEM),
        ],
        out_shape=jax.ShapeDtypeStruct((K, H), _f32),
        scratch_shapes=[pltpu.VMEM((NP, H), _f32)],
    )(scal, g, p, b, aw)


def _head_call(r0, r1, r2, cwt, cb, w1r, b1, w2, b2):
    def body(r0_ref, r1_ref, r2_ref, cwt_ref, cb_ref, w1_ref, b1_ref,
             w2_ref, b2_ref, o_ref):
        conv = (jnp.dot(r0_ref[...], cwt_ref[0:H, :],
                        preferred_element_type=_f32)
                + jnp.dot(r1_ref[...], cwt_ref[H:2 * H, :],
                          preferred_element_type=_f32)
                + jnp.dot(r2_ref[...], cwt_ref[2 * H:3 * H, :],
                          preferred_element_type=_f32)
                + cb_ref[...])
        conv = jnp.where(conv >= 0, conv, conv * _f32(0.01))
        acc = jnp.zeros((1, 128), _f32)
        for i in range(K):
            acc = acc + jnp.dot(conv[i:i + 1, :],
                                w1_ref[i * 16:(i + 1) * 16, :],
                                preferred_element_type=_f32)
        f1 = acc + b1_ref[...]
        f1 = jnp.where(f1 >= 0, f1, f1 * _f32(0.01))
        o = jnp.dot(f1, w2_ref[...], preferred_element_type=_f32) + b2_ref[...]
        o_ref[...] = jax.nn.sigmoid(o)

    return pl.pallas_call(
        body,
        out_shape=jax.ShapeDtypeStruct((1, 2), _f32),
    )(r0, r1, r2, cwt, cb, w1r, b1, w2, b2)


def kernel(x, edge_index, eps0, eps1, eps2, W0, b0, W1, b1, W2, b2,
           aw0, ab0, aw1, ab1, aw2, ab2, convW, convb, fc1W, fc1b, fc2W, fc2b):
    xp = jnp.pad(x, ((0, NP - N), (0, 0)))
    src = edge_index[0]
    dst = edge_index[1]
    npad = EP - E
    src3 = jnp.concatenate(
        [src, jnp.zeros((npad,), jnp.int32)]).reshape(NTILES, NCH, CHK)
    dst3 = jnp.concatenate(
        [dst, jnp.full((npad,), N, jnp.int32)]).reshape(NTILES, NCH, CHK)
    zblk = jnp.zeros((RPT, H), _f32)

    b0r = b0.reshape(1, H)
    b1r = b1.reshape(1, H)
    b2r = b2.reshape(1, H)
    aw0r = aw0.reshape(H, 1)
    aw1r = aw1.reshape(H, 1)
    aw2r = aw2.reshape(H, 1)
    sc0 = jnp.stack([1.0 + eps0, ab0]).astype(_f32)
    sc1 = jnp.stack([1.0 + eps1, ab1]).astype(_f32)
    sc2 = jnp.stack([1.0 + eps2, ab2]).astype(_f32)
    cwt = convW.T
    w1r = fc1W.reshape(16, K, 128).transpose(1, 0, 2).reshape(16 * K, 128)
    cb = convb.reshape(1, 16)
    fb1 = fc1b.reshape(1, 128)
    fb2 = fc2b.reshape(1, 2)

    g0 = _matmul_call(xp, W0)
    p0 = _segsum_sc(g0, src3, dst3, zblk)
    g1, r0 = _combine_call(g0, p0, sc0, b0r, aw0r, W1)
    p1 = _segsum_sc(g1, src3, dst3, zblk)
    g2, r1 = _combine_call(g1, p1, sc1, b1r, aw1r, W2)
    p2 = _segsum_sc(g2, src3, dst3, zblk)
    r2 = _combine_last_call(g2, p2, sc2, b2r, aw2r)

    return _head_call(r0, r1, r2, cwt, cb, w1r, fb1, fc2W, fb2)


# speedup vs baseline: 2.1157x; 2.1157x over previous
import functools

import jax
import jax.numpy as jnp
from jax import lax
from jax.experimental import pallas as pl
from jax.experimental.pallas import tpu as pltpu
from jax.experimental.pallas import tpu_sc as plsc

N = 10000
NP = 10240
D = 128
H = 64
K = 32
E = 320000
NTILES = 32
CHK = 512
NCH = 20
EPT = CHK * NCH
EP = EPT * NTILES
RPT = NP // 16

_f32 = jnp.float32


def _matmul_call(x, w):
    def body(x_ref, w_ref, o_ref):
        o_ref[...] = jnp.dot(x_ref[...], w_ref[...],
                             preferred_element_type=_f32)

    return pl.pallas_call(
        body,
        out_shape=jax.ShapeDtypeStruct((x.shape[0], w.shape[1]), _f32),
    )(x, w)


def _segsum_sc(g, src3, dst3, zblk):
    mesh = plsc.VectorSubcoreMesh(core_axis_name="c", subcore_axis_name="s")

    @functools.partial(
        pl.kernel,
        out_type=jax.ShapeDtypeStruct((2, NP, H), _f32),
        mesh=mesh,
        compiler_params=pltpu.CompilerParams(use_tc_tiling_on_sc=False),
        scratch_types=[
            pltpu.VMEM((CHK,), jnp.int32),
            pltpu.VMEM((CHK,), jnp.int32),
            pltpu.VMEM((CHK, H), _f32),
            pltpu.VMEM_SHARED((NP, H), _f32),
            pltpu.VMEM_SHARED((NP, H), _f32),
        ],
    )
    def ksc(g_hbm, src_hbm, dst_hbm, z_hbm, out_hbm,
            srcv, dstv, rows, acc, gsh):
        c = lax.axis_index("c")
        s = lax.axis_index("s")
        wid = s * 2 + c
        hrpt = RPT // 2

        for u in range(2):
            r0 = s * RPT + u * hrpt
            pltpu.sync_copy(g_hbm.at[pl.ds(r0, hrpt)], rows.at[pl.ds(0, hrpt)])
            pltpu.sync_copy(rows.at[pl.ds(0, hrpt)], gsh.at[pl.ds(r0, hrpt)])
            pltpu.sync_copy(z_hbm.at[pl.ds(0, hrpt)], rows.at[pl.ds(0, hrpt)])
            pltpu.sync_copy(rows.at[pl.ds(0, hrpt)], acc.at[pl.ds(r0, hrpt)])
        plsc.subcore_barrier()

        @pl.loop(0, NCH)
        def _(j):
            pltpu.sync_copy(src_hbm.at[wid, j], srcv)
            pltpu.sync_copy(dst_hbm.at[wid, j], dstv)
            pltpu.sync_copy(gsh.at[srcv], rows)
            pltpu.sync_copy(rows, acc.at[dstv], add=True)

        plsc.subcore_barrier()
        for u in range(2):
            r0 = s * RPT + u * hrpt
            pltpu.sync_copy(acc.at[pl.ds(r0, hrpt)], rows.at[pl.ds(0, hrpt)])
            pltpu.sync_copy(rows.at[pl.ds(0, hrpt)],
                            out_hbm.at[c, pl.ds(r0, hrpt)])

    return ksc(g, src3, dst3, zblk)


def _topk_pool(s80, hs_ref, r_ref):
    ia = lax.broadcasted_iota(jnp.int32, (80, 128), 0)
    ib = lax.broadcasted_iota(jnp.int32, (80, 128), 1)
    fiota = ia * 128 + ib
    s = jnp.where(fiota < N, s80, _f32(-3e38))

    def step(k, sv):
        m = jnp.max(sv)
        fi = jnp.min(jnp.where(sv == m, fiota, jnp.int32(1 << 30)))
        row = hs_ref[pl.ds(fi, 1), :]
        r_ref[pl.ds(k, 1), :] = row * m
        return jnp.where(fiota == fi, _f32(-3e38), sv)

    lax.fori_loop(0, K, step, s)


def _combine_call(g, p, scal, b, aw, wn):
    def body(scal_ref, g_ref, p_ref, b_ref, aw_ref, wn_ref,
             gn_ref, r_ref, hs_ref):
        one_eps = scal_ref[0]
        ab = scal_ref[1]
        h = jnp.maximum(
            one_eps * g_ref[...] + p_ref[0] + p_ref[1] + b_ref[...], 0.0)
        hs_ref[...] = h
        gn_ref[...] = jnp.dot(h, wn_ref[...], preferred_element_type=_f32)
        z = jnp.dot(h, aw_ref[...], preferred_element_type=_f32)
        s80 = jnp.tanh(z + ab).reshape(80, 128)
        _topk_pool(s80, hs_ref, r_ref)

    return pl.pallas_call(
        body,
        in_specs=[
            pl.BlockSpec(memory_space=pltpu.SMEM),
            pl.BlockSpec(memory_space=pltpu.VMEM),
            pl.BlockSpec(memory_space=pltpu.VMEM),
            pl.BlockSpec(memory_space=pltpu.VMEM),
            pl.BlockSpec(memory_space=pltpu.VMEM),
            pl.BlockSpec(memory_space=pltpu.VMEM),
        ],
        out_shape=[
            jax.ShapeDtypeStruct((NP, H), _f32),
            jax.ShapeDtypeStruct((K, H), _f32),
        ],
        scratch_shapes=[pltpu.VMEM((NP, H), _f32)],
    )(scal, g, p, b, aw, wn)


def _combine_last_call(g, p, scal, b, aw):
    def body(scal_ref, g_ref, p_ref, b_ref, aw_ref, r_ref, hs_ref):
        one_eps = scal_ref[0]
        ab = scal_ref[1]
        h = jnp.maximum(
            one_eps * g_ref[...] + p_ref[0] + p_ref[1] + b_ref[...], 0.0)
        hs_ref[...] = h
        z = jnp.dot(h, aw_ref[...], preferred_element_type=_f32)
        s80 = jnp.tanh(z + ab).reshape(80, 128)
        _topk_pool(s80, hs_ref, r_ref)

    return pl.pallas_call(
        body,
        in_specs=[
            pl.BlockSpec(memory_space=pltpu.SMEM),
            pl.BlockSpec(memory_space=pltpu.VMEM),
            pl.BlockSpec(memory_space=pltpu.VMEM),
            pl.BlockSpec(memory_space=pltpu.VMEM),
            pl.BlockSpec(memory_space=pltpu.VMEM),
        ],
        out_shape=jax.ShapeDtypeStruct((K, H), _f32),
        scratch_shapes=[pltpu.VMEM((NP, H), _f32)],
    )(scal, g, p, b, aw)


def _head_call(r0, r1, r2, cwt, cb, w1r, b1, w2, b2):
    def body(r0_ref, r1_ref, r2_ref, cwt_ref, cb_ref, w1_ref, b1_ref,
             w2_ref, b2_ref, o_ref):
        conv = (jnp.dot(r0_ref[...], cwt_ref[0:H, :],
                        preferred_element_type=_f32)
                + jnp.dot(r1_ref[...], cwt_ref[H:2 * H, :],
                          preferred_element_type=_f32)
                + jnp.dot(r2_ref[...], cwt_ref[2 * H:3 * H, :],
                          preferred_element_type=_f32)
                + cb_ref[...])
        conv = jnp.where(conv >= 0, conv, conv * _f32(0.01))
        acc = jnp.zeros((1, 128), _f32)
        for i in range(K):
            acc = acc + jnp.dot(conv[i:i + 1, :],
                                w1_ref[i * 16:(i + 1) * 16, :],
                                preferred_element_type=_f32)
        f1 = acc + b1_ref[...]
        f1 = jnp.where(f1 >= 0, f1, f1 * _f32(0.01))
        o = jnp.dot(f1, w2_ref[...], preferred_element_type=_f32) + b2_ref[...]
        o_ref[...] = jax.nn.sigmoid(o)

    return pl.pallas_call(
        body,
        out_shape=jax.ShapeDtypeStruct((1, 2), _f32),
    )(r0, r1, r2, cwt, cb, w1r, b1, w2, b2)


def kernel(x, edge_index, eps0, eps1, eps2, W0, b0, W1, b1, W2, b2,
           aw0, ab0, aw1, ab1, aw2, ab2, convW, convb, fc1W, fc1b, fc2W, fc2b):
    xp = jnp.pad(x, ((0, NP - N), (0, 0)))
    src = edge_index[0]
    dst = edge_index[1]
    npad = EP - E
    src3 = jnp.concatenate(
        [src, jnp.zeros((npad,), jnp.int32)]).reshape(NTILES, NCH, CHK)
    dst3 = jnp.concatenate(
        [dst, jnp.full((npad,), N, jnp.int32)]).reshape(NTILES, NCH, CHK)

    zblk = jnp.zeros((RPT, H), _f32)

    b0r = b0.reshape(1, H)
    b1r = b1.reshape(1, H)
    b2r = b2.reshape(1, H)
    aw0r = aw0.reshape(H, 1)
    aw1r = aw1.reshape(H, 1)
    aw2r = aw2.reshape(H, 1)
    sc0 = jnp.stack([1.0 + eps0, ab0]).astype(_f32)
    sc1 = jnp.stack([1.0 + eps1, ab1]).astype(_f32)
    sc2 = jnp.stack([1.0 + eps2, ab2]).astype(_f32)
    cwt = convW.T
    w1r = fc1W.reshape(16, K, 128).transpose(1, 0, 2).reshape(16 * K, 128)
    cb = convb.reshape(1, 16)
    fb1 = fc1b.reshape(1, 128)
    fb2 = fc2b.reshape(1, 2)

    g0 = _matmul_call(xp, W0)
    p0 = _segsum_sc(g0, src3, dst3, zblk)
    g1, r0 = _combine_call(g0, p0, sc0, b0r, aw0r, W1)
    p1 = _segsum_sc(g1, src3, dst3, zblk)
    g2, r1 = _combine_call(g1, p1, sc1, b1r, aw1r, W2)
    p2 = _segsum_sc(g2, src3, dst3, zblk)
    r2 = _combine_last_call(g2, p2, sc2, b2r, aw2r)

    return _head_call(r0, r1, r2, cwt, cb, w1r, fb1, fc2W, fb2)
